# baseline (speedup 1.0000x reference)
import functools
import jax
import jax.numpy as jnp
from jax import lax
from jax.experimental import pallas as pl
from jax.experimental.pallas import tpu as pltpu
from jax.experimental.pallas import tpu_sc as plsc

N = 10000
E = 320000
HE = 150000
IN_C = 128
HID = 128
OUT_C = 64
MASK_RATIO = 0.3
EDGE_DROP_RATIO = 0.2
T_DIFFUSION = 20
BETA_START = 0.0001
BETA_END = 0.02
GAMMA = 0.8
TEMPERATURE = 0.7

NC = 2
NS = 16
NW = NC * NS
K = 128
DUMMY = N
MC = 160
NACC_P = 10080
RPS_P = NACC_P // NS
ZR_P = 63
NACC_C = 10016
RPS_C = NACC_C // NS
ZR_C = 313
NBUF = 4

ROW_BLK = 1000
COL_BLK = 1000


def _pad_edges(idx, total, fill):
    pad = total - idx.shape[0]
    return jnp.concatenate([idx, jnp.full((pad,), fill, jnp.int32)])


def _zero_acc_slice(zbuf_v, acc_sh, s, F, rps, zr):
    def zrow(i, carry):
        for j in range(F // 16):
            zbuf_v[i, pl.ds(j * 16, 16)] = jnp.zeros((16,), jnp.float32)
        return carry

    lax.fori_loop(0, zr, zrow, 0)

    def zacc(i, carry):
        pltpu.sync_copy(zbuf_v, acc_sh.at[pl.ds(s * rps + i * zr, zr)])
        return carry

    lax.fori_loop(0, rps // zr, zacc, 0)


def _writeout(zbuf_v, acc_sh, out_hbm, c, s, rps, zr):
    def wout(i, carry):
        r0 = s * rps + i * zr
        pltpu.sync_copy(acc_sh.at[pl.ds(r0, zr)], zbuf_v)
        pltpu.sync_copy(zbuf_v, out_hbm.at[c, pl.ds(r0, zr)])
        return carry

    lax.fori_loop(0, rps // zr, wout, 0)


@functools.lru_cache(maxsize=None)
def _make_propagate(F, n_table):
    mesh = plsc.VectorSubcoreMesh(core_axis_name="c", subcore_axis_name="s")

    @functools.partial(
        pl.kernel,
        out_type=jax.ShapeDtypeStruct((NC, NACC_P, F), jnp.float32),
        mesh=mesh,
        compiler_params=pltpu.CompilerParams(use_tc_tiling_on_sc=False),
        scratch_types=[
            pltpu.VMEM((16,), jnp.int32),
            pltpu.VMEM((MC, K), jnp.int32),
            pltpu.VMEM((MC, K), jnp.int32),
            pltpu.VMEM((2, K, F), jnp.float32),
            pltpu.VMEM((ZR_P, F), jnp.float32),
            pltpu.VMEM_SHARED((NACC_P, F), jnp.float32),
            pltpu.SemaphoreType.DMA,
            pltpu.SemaphoreType.DMA,
        ],
    )
    def prop(tok_hbm, meta_hbm, table_hbm, sidx_hbm, didx_hbm, out_hbm,
             meta_v, sidx_v, didx_v, rows_v, zbuf_v, acc_sh, sem0, sem1):
        c = lax.axis_index("c")
        s = lax.axis_index("s")
        wid = s * NC + c

        pltpu.sync_copy(tok_hbm, zbuf_v.at[0, pl.ds(0, 16)])
        pltpu.sync_copy(meta_hbm, meta_v)
        nchunk = meta_v[...][0]
        _zero_acc_slice(zbuf_v, acc_sh, s, F, RPS_P, ZR_P)
        pltpu.sync_copy(sidx_hbm.at[wid], sidx_v)
        pltpu.sync_copy(didx_hbm.at[wid], didx_v)
        plsc.subcore_barrier()

        sems = (sem0, sem1)
        pltpu.async_copy(table_hbm.at[sidx_v.at[0]], rows_v.at[0], sem0)
        pltpu.async_copy(table_hbm.at[sidx_v.at[1]], rows_v.at[1], sem1)

        def pair(i, carry):
            ci0 = 2 * i
            for b in range(2):
                ci = ci0 + b
                pltpu.make_async_copy(table_hbm.at[sidx_v.at[ci]],
                                      rows_v.at[b], sems[b]).wait()
                pltpu.sync_copy(rows_v.at[b], acc_sh.at[didx_v.at[ci]], add=True)
                nxt = jnp.minimum(ci + 2, nchunk - 1)

                @pl.when(ci + 2 < nchunk)
                def _():
                    pltpu.async_copy(table_hbm.at[sidx_v.at[nxt]],
                                     rows_v.at[b], sems[b])
            return carry

        lax.fori_loop(0, nchunk // 2, pair, 0)
        plsc.subcore_barrier()
        _writeout(zbuf_v, acc_sh, out_hbm, c, s, RPS_P, ZR_P)

    return prop


CF = 16


@functools.lru_cache(maxsize=None)
def _make_count():
    mesh = plsc.VectorSubcoreMesh(core_axis_name="c", subcore_axis_name="s")

    @functools.partial(
        pl.kernel,
        out_type=jax.ShapeDtypeStruct((NC, NACC_C, CF), jnp.float32),
        mesh=mesh,
        compiler_params=pltpu.CompilerParams(use_tc_tiling_on_sc=False),
        scratch_types=[
            pltpu.VMEM((16,), jnp.int32),
            pltpu.VMEM((MC, K), jnp.int32),
            pltpu.VMEM((K, CF), jnp.float32),
            pltpu.VMEM((ZR_C, CF), jnp.float32),
            pltpu.VMEM_SHARED((NACC_C, CF), jnp.float32),
        ],
    )
    def cnt(tok_hbm, meta_hbm, didx_hbm, out_hbm,
            meta_v, didx_v, ones_v, zbuf_v, acc_sh):
        c = lax.axis_index("c")
        s = lax.axis_index("s")
        wid = s * NC + c

        pltpu.sync_copy(tok_hbm, zbuf_v.at[0, pl.ds(0, 16)])
        pltpu.sync_copy(meta_hbm, meta_v)
        nchunk = meta_v[...][0]
        _zero_acc_slice(zbuf_v, acc_sh, s, CF, RPS_C, ZR_C)

        def orow(i, carry):
            ones_v[i, pl.ds(0, 16)] = jnp.ones((16,), jnp.float32)
            return carry

        lax.fori_loop(0, K, orow, 0)
        pltpu.sync_copy(didx_hbm.at[wid], didx_v)
        plsc.subcore_barrier()

        def chunk(ci, carry):
            pltpu.sync_copy(ones_v, acc_sh.at[didx_v.at[ci]], add=True)
            return carry

        lax.fori_loop(0, nchunk, chunk, 0)
        plsc.subcore_barrier()
        _writeout(zbuf_v, acc_sh, out_hbm, c, s, RPS_C, ZR_C)

    return cnt


def _shape_idx(flat, nchunk):
    a = flat.reshape(NW, nchunk, K)
    if nchunk < MC:
        a = jnp.concatenate(
            [a, jnp.zeros((NW, MC - nchunk, K), jnp.int32)], axis=1)
    return a


FP = 64


def _propagate(table, sidx, didx, nchunk, tok):
    assert table.shape[1] == FP
    meta = jnp.full((16,), nchunk, jnp.int32)
    parts = _make_propagate(FP, table.shape[0])(tok, meta, table, sidx, didx)
    return parts[0, :N] + parts[1, :N], parts[0, 0, :16]


def _propagate_wide(table, sidx, didx, nchunk, tok):
    halves = []
    for f0 in range(0, table.shape[1], FP):
        h, tok = _propagate(table[:, f0:f0 + FP], sidx, didx, nchunk, tok)
        halves.append(h)
    return jnp.concatenate(halves, axis=1) if len(halves) > 1 else halves[0], tok


def _count(didx, nchunk, tok):
    meta = jnp.full((16,), nchunk, jnp.int32)
    parts = _make_count()(tok, meta, didx)
    return parts[0, :N, 0] + parts[1, :N, 0], parts[0, 0, :16]


def _contrastive_body(z1_ref, z2_ref, out_ref):
    r = pl.program_id(0)
    z1 = z1_ref[...]
    inv_t = 1.0 / TEMPERATURE
    row_ids = r * ROW_BLK + jax.lax.broadcasted_iota(jnp.int32, (ROW_BLK, COL_BLK), 0)

    def step(c, carry):
        m, s, pos = carry
        z2c = z2_ref[pl.ds(c * COL_BLK, COL_BLK), :]
        sim = jax.lax.dot_general(z1, z2c, (((1,), (1,)), ((), ())),
                                  preferred_element_type=jnp.float32) * inv_t
        col_ids = c * COL_BLK + jax.lax.broadcasted_iota(jnp.int32, (ROW_BLK, COL_BLK), 1)
        diag = row_ids == col_ids
        pos = pos + jnp.sum(jnp.where(diag, sim, 0.0), axis=1, keepdims=True)
        simm = jnp.where(diag, -jnp.inf, sim)
        m_new = jnp.maximum(m, jnp.max(simm, axis=1, keepdims=True))
        s = s * jnp.exp(m - m_new) + jnp.sum(jnp.exp(simm - m_new), axis=1, keepdims=True)
        return m_new, s, pos

    m0 = jnp.full((ROW_BLK, 1), -jnp.inf, jnp.float32)
    s0 = jnp.zeros((ROW_BLK, 1), jnp.float32)
    p0 = jnp.zeros((ROW_BLK, 1), jnp.float32)
    m, s, pos = jax.lax.fori_loop(0, N // COL_BLK, step, (m0, s0, p0))
    out_ref[...] = pos - (m + jnp.log(s))


def _contrastive_loss_pallas(z1, z2):
    z1 = z1 / jnp.maximum(jnp.linalg.norm(z1, axis=1, keepdims=True), 1e-12)
    z2 = z2 / jnp.maximum(jnp.linalg.norm(z2, axis=1, keepdims=True), 1e-12)
    per_row = pl.pallas_call(
        _contrastive_body,
        grid=(N // ROW_BLK,),
        in_specs=[
            pl.BlockSpec((ROW_BLK, OUT_C), lambda r: (r, 0)),
            pl.BlockSpec((N, OUT_C), lambda r: (0, 0)),
        ],
        out_specs=pl.BlockSpec((ROW_BLK, 1), lambda r: (r, 0)),
        out_shape=jax.ShapeDtypeStruct((N, 1), jnp.float32),
    )(z1, z2)
    return -jnp.mean(per_row)


def _layer_norm(h, g, b, eps=1e-5):
    mu = jnp.mean(h, axis=-1, keepdims=True)
    var = jnp.mean((h - mu) ** 2, axis=-1, keepdims=True)
    return (h - mu) / jnp.sqrt(var + eps) * g + b


def kernel(x, edge_index, hyperedge_index, params):
    p = params
    key = jax.random.key(42)
    k1, k2, k3 = jax.random.split(key, 3)
    src, dst = edge_index[0], edge_index[1]
    mask = (jax.random.uniform(k1, x.shape) > MASK_RATIO).astype(x.dtype)
    x_bar = x * mask
    keep = (jax.random.uniform(k2, (src.shape[0],)) >= EDGE_DROP_RATIO) & (src <= dst)

    n = x.shape[0]
    loop = jnp.arange(n, dtype=src.dtype)
    s2 = jnp.concatenate([src, dst, loop])
    d2_raw = jnp.concatenate([dst, src, loop])
    valid = jnp.concatenate([keep, keep, jnp.ones((n,), bool)])
    d2 = jnp.where(valid, d2_raw, DUMMY)

    E2 = 2 * E + N
    nchunk_g = 2 * (-(-E2 // (2 * NW * K)))
    E2P = NW * K * nchunk_g
    s2f = _shape_idx(_pad_edges(s2, E2P, 0), nchunk_g)
    d2f = _shape_idx(_pad_edges(d2, E2P, DUMMY), nchunk_g)

    nid, hid_ = hyperedge_index[0], hyperedge_index[1]
    nchunk_h = 8 * (-(-HE // (8 * NW * K)))
    HEP = NW * K * nchunk_h
    nidf = _shape_idx(_pad_edges(nid, HEP, 0), nchunk_h)
    hidf = _shape_idx(_pad_edges(hid_, HEP, DUMMY), nchunk_h)
    nidf_d = _shape_idx(_pad_edges(nid, HEP, DUMMY), nchunk_h)

    tok = jnp.zeros((16,), jnp.float32)
    deg, tok = _count(d2f, nchunk_g, tok)
    Bc, tok = _count(hidf, nchunk_h, tok)
    Dc, tok = _count(nidf_d, nchunk_h, tok)
    dis = jnp.where(deg > 0, jax.lax.rsqrt(deg), 0.0)
    Binv = jnp.where(Bc > 0, 1.0 / Bc, 0.0)
    Dinv = jnp.where(Dc > 0, 1.0 / Dc, 0.0)

    def gcn_agg(h_in):
        nonlocal tok
        P, tok = _propagate_wide(dis[:, None] * h_in, s2f, d2f, nchunk_g, tok)
        return dis[:, None] * P

    def lnrelu(z, g, b):
        return jax.nn.relu(_layer_norm(z, g, b))

    Q1 = gcn_agg(x_bar)

    h = lnrelu(Q1 @ p['ex_gcn1_W'] + p['ex_gcn1_b'], p['ex_ln1_g'], p['ex_ln1_b'])
    h = lnrelu(gcn_agg(h) @ p['ex_gcn2_W'] + p['ex_gcn2_b'], p['ex_ln1_g'], p['ex_ln1_b'])
    h_x = jax.nn.relu(h @ p['ex_p1_W'] + p['ex_p1_b']) @ p['ex_p2_W'] + p['ex_p2_b']

    h1 = lnrelu(Q1 @ p['ey_gcn1_W'] + p['ey_gcn1_b'], p['ey_ln1_g'], p['ey_ln1_b'])
    h1 = lnrelu(gcn_agg(h1) @ p['ey_gcn2_W'] + p['ey_gcn2_b'], p['ey_ln1_g'], p['ey_ln1_b'])

    def hgc_agg(h_in):
        nonlocal tok
        M1, tok = _propagate_wide(h_in, nidf, hidf, nchunk_h, tok)
        P2, tok = _propagate_wide(Binv[:, None] * M1, hidf, nidf_d, nchunk_h, tok)
        return Dinv[:, None] * P2

    h2 = lnrelu(hgc_agg(x_bar) @ p['ey_hgc1_W'] + p['ey_hgc1_b'], p['ey_ln2_g'], p['ey_ln2_b'])
    h2 = lnrelu(hgc_agg(h2) @ p['ey_hgc2_W'] + p['ey_hgc2_b'], p['ey_ln2_g'], p['ey_ln2_b'])
    hy = (h1 + h2) / 2.0
    h_y = jax.nn.relu(hy @ p['ey_p1_W'] + p['ey_p1_b']) @ p['ey_p2_W'] + p['ey_p2_b']

    loss_c = _contrastive_loss_pallas(h_x, h_y)

    t = 10
    beta = jnp.linspace(BETA_START, BETA_END, T_DIFFUSION)
    alpha_cum = jnp.cumprod(1.0 - beta)
    sa = jnp.sqrt(alpha_cum[t])
    so = jnp.sqrt(1.0 - alpha_cum[t])
    noise = jax.random.normal(k3, h_x.shape, dtype=h_x.dtype)
    h_noisy = sa * h_x + so * noise
    t_in = jnp.array([[t / T_DIFFUSION]], dtype=h_x.dtype)
    t_emb = jax.nn.relu(t_in @ p['dn_t1_W'] + p['dn_t1_b']) @ p['dn_t2_W'] + p['dn_t2_b']
    hn = h_noisy + t_emb
    hn = lnrelu(gcn_agg(hn) @ p['dn_c1_W'] + p['dn_c1_b'], p['dn_ln_g'], p['dn_ln_b'])
    h_hat = gcn_agg(hn) @ p['dn_c2_W'] + p['dn_c2_b']
    loss_g = jnp.mean((h_hat - h_x) ** 2)
    loss = GAMMA * loss_c + (1.0 - GAMMA) * loss_g
    return loss, jax.lax.stop_gradient(h_x)


# speedup vs baseline: 1.3009x; 1.3009x over previous
import functools
import jax
import jax.numpy as jnp
from jax import lax
from jax.experimental import pallas as pl
from jax.experimental.pallas import tpu as pltpu
from jax.experimental.pallas import tpu_sc as plsc

N = 10000
E = 320000
HE = 150000
IN_C = 128
HID = 128
OUT_C = 64
MASK_RATIO = 0.3
EDGE_DROP_RATIO = 0.2
T_DIFFUSION = 20
BETA_START = 0.0001
BETA_END = 0.02
GAMMA = 0.8
TEMPERATURE = 0.7

NC = 2
NS = 16
NW = NC * NS
K = 128
DUMMY = N
MC = 160
NACC_P = 10080
RPS_P = NACC_P // NS
ZR_P = 63
NACC_C = 10016
RPS_C = NACC_C // NS
ZR_C = 313
NBUF = 4

ROW_BLK = 1000
COL_BLK = 1000


def _pad_edges(idx, total, fill):
    pad = total - idx.shape[0]
    return jnp.concatenate([idx, jnp.full((pad,), fill, jnp.int32)])


def _zero_acc_slice(zbuf_v, acc_sh, s, F, rps, zr):
    def zrow(i, carry):
        for j in range(F // 16):
            zbuf_v[i, pl.ds(j * 16, 16)] = jnp.zeros((16,), jnp.float32)
        return carry

    lax.fori_loop(0, zr, zrow, 0)

    def zacc(i, carry):
        pltpu.sync_copy(zbuf_v, acc_sh.at[pl.ds(s * rps + i * zr, zr)])
        return carry

    lax.fori_loop(0, rps // zr, zacc, 0)


def _writeout(zbuf_v, acc_sh, out_hbm, c, s, rps, zr):
    def wout(i, carry):
        r0 = s * rps + i * zr
        pltpu.sync_copy(acc_sh.at[pl.ds(r0, zr)], zbuf_v)
        pltpu.sync_copy(zbuf_v, out_hbm.at[c, pl.ds(r0, zr)])
        return carry

    lax.fori_loop(0, rps // zr, wout, 0)


@functools.lru_cache(maxsize=None)
def _make_propagate(F, nchunk, n_table):
    mesh = plsc.VectorSubcoreMesh(core_axis_name="c", subcore_axis_name="s")

    @functools.partial(
        pl.kernel,
        out_type=jax.ShapeDtypeStruct((NC, NACC_P, F), jnp.float32),
        mesh=mesh,
        compiler_params=pltpu.CompilerParams(use_tc_tiling_on_sc=False),
        scratch_types=[
            pltpu.VMEM((nchunk, K), jnp.int32),
            pltpu.VMEM((nchunk, K), jnp.int32),
            pltpu.VMEM((2, K, F), jnp.float32),
            pltpu.VMEM((ZR_P, F), jnp.float32),
            pltpu.VMEM_SHARED((NACC_P, F), jnp.float32),
            pltpu.SemaphoreType.DMA,
            pltpu.SemaphoreType.DMA,
        ],
    )
    def prop(tok_hbm, table_hbm, sidx_hbm, didx_hbm, out_hbm,
             sidx_v, didx_v, rows_v, zbuf_v, acc_sh, sem0, sem1):
        c = lax.axis_index("c")
        s = lax.axis_index("s")
        wid = s * NC + c

        pltpu.sync_copy(tok_hbm, zbuf_v.at[0, pl.ds(0, 16)])
        _zero_acc_slice(zbuf_v, acc_sh, s, F, RPS_P, ZR_P)
        pltpu.sync_copy(sidx_hbm.at[wid], sidx_v)
        pltpu.sync_copy(didx_hbm.at[wid], didx_v)
        plsc.subcore_barrier()

        sems = (sem0, sem1)
        pltpu.async_copy(table_hbm.at[sidx_v.at[0]], rows_v.at[0], sem0)
        pltpu.async_copy(table_hbm.at[sidx_v.at[1]], rows_v.at[1], sem1)

        def pair(i, carry):
            ci0 = 2 * i
            for b in range(2):
                ci = ci0 + b
                pltpu.make_async_copy(table_hbm.at[sidx_v.at[ci]],
                                      rows_v.at[b], sems[b]).wait()
                pltpu.sync_copy(rows_v.at[b], acc_sh.at[didx_v.at[ci]], add=True)
                nxt = jnp.minimum(ci + 2, nchunk - 1)

                @pl.when(ci + 2 < nchunk)
                def _():
                    pltpu.async_copy(table_hbm.at[sidx_v.at[nxt]],
                                     rows_v.at[b], sems[b])
            return carry

        lax.fori_loop(0, nchunk // 2, pair, 0)
        plsc.subcore_barrier()
        _writeout(zbuf_v, acc_sh, out_hbm, c, s, RPS_P, ZR_P)

    return prop


CF = 16


@functools.lru_cache(maxsize=None)
def _make_count(nchunk):
    mesh = plsc.VectorSubcoreMesh(core_axis_name="c", subcore_axis_name="s")

    @functools.partial(
        pl.kernel,
        out_type=jax.ShapeDtypeStruct((NC, NACC_C, CF), jnp.float32),
        mesh=mesh,
        compiler_params=pltpu.CompilerParams(use_tc_tiling_on_sc=False),
        scratch_types=[
            pltpu.VMEM((nchunk, K), jnp.int32),
            pltpu.VMEM((K, CF), jnp.float32),
            pltpu.VMEM((ZR_C, CF), jnp.float32),
            pltpu.VMEM_SHARED((NACC_C, CF), jnp.float32),
        ],
    )
    def cnt(tok_hbm, didx_hbm, out_hbm,
            didx_v, ones_v, zbuf_v, acc_sh):
        c = lax.axis_index("c")
        s = lax.axis_index("s")
        wid = s * NC + c

        pltpu.sync_copy(tok_hbm, zbuf_v.at[0, pl.ds(0, 16)])
        _zero_acc_slice(zbuf_v, acc_sh, s, CF, RPS_C, ZR_C)

        def orow(i, carry):
            ones_v[i, pl.ds(0, 16)] = jnp.ones((16,), jnp.float32)
            return carry

        lax.fori_loop(0, K, orow, 0)
        pltpu.sync_copy(didx_hbm.at[wid], didx_v)
        plsc.subcore_barrier()

        def chunk(ci, carry):
            pltpu.sync_copy(ones_v, acc_sh.at[didx_v.at[ci]], add=True)
            return carry

        lax.fori_loop(0, nchunk, chunk, 0)
        plsc.subcore_barrier()
        _writeout(zbuf_v, acc_sh, out_hbm, c, s, RPS_C, ZR_C)

    return cnt


def _shape_idx(flat, nchunk):
    return flat.reshape(NW, nchunk, K)


FP = 64


def _propagate(table, sidx, didx, nchunk, tok):
    assert table.shape[1] == FP
    parts = _make_propagate(FP, nchunk, table.shape[0])(tok, table, sidx, didx)
    return parts[0, :N] + parts[1, :N], parts[0, 0, :16]


def _propagate_wide(table, sidx, didx, nchunk, tok):
    halves = []
    for f0 in range(0, table.shape[1], FP):
        h, tok = _propagate(table[:, f0:f0 + FP], sidx, didx, nchunk, tok)
        halves.append(h)
    return jnp.concatenate(halves, axis=1) if len(halves) > 1 else halves[0], tok


def _count(didx, nchunk, tok):
    parts = _make_count(nchunk)(tok, didx)
    return parts[0, :N, 0] + parts[1, :N, 0], parts[0, 0, :16]


def _contrastive_body(z1_ref, z2_ref, out_ref):
    r = pl.program_id(0)
    z1 = z1_ref[...]
    inv_t = 1.0 / TEMPERATURE
    row_ids = r * ROW_BLK + jax.lax.broadcasted_iota(jnp.int32, (ROW_BLK, COL_BLK), 0)

    def step(c, carry):
        m, s, pos = carry
        z2c = z2_ref[pl.ds(c * COL_BLK, COL_BLK), :]
        sim = jax.lax.dot_general(z1, z2c, (((1,), (1,)), ((), ())),
                                  preferred_element_type=jnp.float32) * inv_t
        col_ids = c * COL_BLK + jax.lax.broadcasted_iota(jnp.int32, (ROW_BLK, COL_BLK), 1)
        diag = row_ids == col_ids
        pos = pos + jnp.sum(jnp.where(diag, sim, 0.0), axis=1, keepdims=True)
        simm = jnp.where(diag, -jnp.inf, sim)
        m_new = jnp.maximum(m, jnp.max(simm, axis=1, keepdims=True))
        s = s * jnp.exp(m - m_new) + jnp.sum(jnp.exp(simm - m_new), axis=1, keepdims=True)
        return m_new, s, pos

    m0 = jnp.full((ROW_BLK, 1), -jnp.inf, jnp.float32)
    s0 = jnp.zeros((ROW_BLK, 1), jnp.float32)
    p0 = jnp.zeros((ROW_BLK, 1), jnp.float32)
    m, s, pos = jax.lax.fori_loop(0, N // COL_BLK, step, (m0, s0, p0))
    out_ref[...] = pos - (m + jnp.log(s))


def _contrastive_loss_pallas(z1, z2):
    z1 = z1 / jnp.maximum(jnp.linalg.norm(z1, axis=1, keepdims=True), 1e-12)
    z2 = z2 / jnp.maximum(jnp.linalg.norm(z2, axis=1, keepdims=True), 1e-12)
    per_row = pl.pallas_call(
        _contrastive_body,
        grid=(N // ROW_BLK,),
        in_specs=[
            pl.BlockSpec((ROW_BLK, OUT_C), lambda r: (r, 0)),
            pl.BlockSpec((N, OUT_C), lambda r: (0, 0)),
        ],
        out_specs=pl.BlockSpec((ROW_BLK, 1), lambda r: (r, 0)),
        out_shape=jax.ShapeDtypeStruct((N, 1), jnp.float32),
    )(z1, z2)
    return -jnp.mean(per_row)


def _layer_norm(h, g, b, eps=1e-5):
    mu = jnp.mean(h, axis=-1, keepdims=True)
    var = jnp.mean((h - mu) ** 2, axis=-1, keepdims=True)
    return (h - mu) / jnp.sqrt(var + eps) * g + b


def kernel(x, edge_index, hyperedge_index, params):
    p = params
    key = jax.random.key(42)
    k1, k2, k3 = jax.random.split(key, 3)
    src, dst = edge_index[0], edge_index[1]
    mask = (jax.random.uniform(k1, x.shape) > MASK_RATIO).astype(x.dtype)
    x_bar = x * mask
    keep = (jax.random.uniform(k2, (src.shape[0],)) >= EDGE_DROP_RATIO) & (src <= dst)

    n = x.shape[0]
    loop = jnp.arange(n, dtype=src.dtype)
    s2 = jnp.concatenate([src, dst, loop])
    d2_raw = jnp.concatenate([dst, src, loop])
    valid = jnp.concatenate([keep, keep, jnp.ones((n,), bool)])
    d2 = jnp.where(valid, d2_raw, DUMMY)

    E2 = 2 * E + N
    nchunk_g = 2 * (-(-E2 // (2 * NW * K)))
    E2P = NW * K * nchunk_g
    s2f = _shape_idx(_pad_edges(s2, E2P, 0), nchunk_g)
    d2f = _shape_idx(_pad_edges(d2, E2P, DUMMY), nchunk_g)

    nid, hid_ = hyperedge_index[0], hyperedge_index[1]
    nchunk_h = 2 * (-(-HE // (2 * NW * K)))
    HEP = NW * K * nchunk_h
    nidf = _shape_idx(_pad_edges(nid, HEP, 0), nchunk_h)
    hidf = _shape_idx(_pad_edges(hid_, HEP, DUMMY), nchunk_h)
    nidf_d = _shape_idx(_pad_edges(nid, HEP, DUMMY), nchunk_h)

    tok = jnp.zeros((16,), jnp.float32)
    deg, tok = _count(d2f, nchunk_g, tok)
    Bc, tok = _count(hidf, nchunk_h, tok)
    Dc, tok = _count(nidf_d, nchunk_h, tok)
    dis = jnp.where(deg > 0, jax.lax.rsqrt(deg), 0.0)
    Binv = jnp.where(Bc > 0, 1.0 / Bc, 0.0)
    Dinv = jnp.where(Dc > 0, 1.0 / Dc, 0.0)

    def gcn_agg(h_in):
        nonlocal tok
        P, tok = _propagate_wide(dis[:, None] * h_in, s2f, d2f, nchunk_g, tok)
        return dis[:, None] * P

    def lnrelu(z, g, b):
        return jax.nn.relu(_layer_norm(z, g, b))

    Q1 = gcn_agg(x_bar)

    h = lnrelu(Q1 @ p['ex_gcn1_W'] + p['ex_gcn1_b'], p['ex_ln1_g'], p['ex_ln1_b'])
    h = lnrelu(gcn_agg(h) @ p['ex_gcn2_W'] + p['ex_gcn2_b'], p['ex_ln1_g'], p['ex_ln1_b'])
    h_x = jax.nn.relu(h @ p['ex_p1_W'] + p['ex_p1_b']) @ p['ex_p2_W'] + p['ex_p2_b']

    h1 = lnrelu(Q1 @ p['ey_gcn1_W'] + p['ey_gcn1_b'], p['ey_ln1_g'], p['ey_ln1_b'])
    h1 = lnrelu(gcn_agg(h1) @ p['ey_gcn2_W'] + p['ey_gcn2_b'], p['ey_ln1_g'], p['ey_ln1_b'])

    def hgc_agg(h_in):
        nonlocal tok
        M1, tok = _propagate_wide(h_in, nidf, hidf, nchunk_h, tok)
        P2, tok = _propagate_wide(Binv[:, None] * M1, hidf, nidf_d, nchunk_h, tok)
        return Dinv[:, None] * P2

    h2 = lnrelu(hgc_agg(x_bar) @ p['ey_hgc1_W'] + p['ey_hgc1_b'], p['ey_ln2_g'], p['ey_ln2_b'])
    h2 = lnrelu(hgc_agg(h2) @ p['ey_hgc2_W'] + p['ey_hgc2_b'], p['ey_ln2_g'], p['ey_ln2_b'])
    hy = (h1 + h2) / 2.0
    h_y = jax.nn.relu(hy @ p['ey_p1_W'] + p['ey_p1_b']) @ p['ey_p2_W'] + p['ey_p2_b']

    loss_c = _contrastive_loss_pallas(h_x, h_y)

    t = 10
    beta = jnp.linspace(BETA_START, BETA_END, T_DIFFUSION)
    alpha_cum = jnp.cumprod(1.0 - beta)
    sa = jnp.sqrt(alpha_cum[t])
    so = jnp.sqrt(1.0 - alpha_cum[t])
    noise = jax.random.normal(k3, h_x.shape, dtype=h_x.dtype)
    h_noisy = sa * h_x + so * noise
    t_in = jnp.array([[t / T_DIFFUSION]], dtype=h_x.dtype)
    t_emb = jax.nn.relu(t_in @ p['dn_t1_W'] + p['dn_t1_b']) @ p['dn_t2_W'] + p['dn_t2_b']
    hn = h_noisy + t_emb
    hn = lnrelu(gcn_agg(hn) @ p['dn_c1_W'] + p['dn_c1_b'], p['dn_ln_g'], p['dn_ln_b'])
    h_hat = gcn_agg(hn) @ p['dn_c2_W'] + p['dn_c2_b']
    loss_g = jnp.mean((h_hat - h_x) ** 2)
    loss = GAMMA * loss_c + (1.0 - GAMMA) * loss_g
    return loss, jax.lax.stop_gradient(h_x)


# speedup vs baseline: 1.6981x; 1.3053x over previous
import functools
import jax
import jax.numpy as jnp
from jax import lax
from jax.experimental import pallas as pl
from jax.experimental.pallas import tpu as pltpu
from jax.experimental.pallas import tpu_sc as plsc

N = 10000
E = 320000
HE = 150000
IN_C = 128
HID = 128
OUT_C = 64
MASK_RATIO = 0.3
EDGE_DROP_RATIO = 0.2
T_DIFFUSION = 20
BETA_START = 0.0001
BETA_END = 0.02
GAMMA = 0.8
TEMPERATURE = 0.7

NC = 2
NS = 16
NW = NC * NS
K = 128
DUMMY = N
DSPREAD = 2000
NACC_P = 12000
RPS_P = NACC_P // NS
ZR_P = 75
NACC_C = 12000
RPS_C = NACC_C // NS
ZR_C = 75

ROW_BLK = 1000
COL_BLK = 1000


def _pad_edges(idx, total, fill):
    pad = total - idx.shape[0]
    return jnp.concatenate([idx, jnp.full((pad,), fill, jnp.int32)])


def _pad_didx(idx, total):
    pad = total - idx.shape[0]
    fills = DUMMY + (jnp.arange(pad, dtype=jnp.int32) % DSPREAD)
    return jnp.concatenate([idx, fills])


def _zero_acc_slice(zbuf_v, acc_sh, s, F, rps, zr):
    def zrow(i, carry):
        for j in range(F // 16):
            zbuf_v[i, pl.ds(j * 16, 16)] = jnp.zeros((16,), jnp.float32)
        return carry

    lax.fori_loop(0, zr, zrow, 0)

    def zacc(i, carry):
        pltpu.sync_copy(zbuf_v, acc_sh.at[pl.ds(s * rps + i * zr, zr)])
        return carry

    lax.fori_loop(0, rps // zr, zacc, 0)


def _writeout(zbuf_v, acc_sh, out_hbm, c, s, rps, zr):
    def wout(i, carry):
        r0 = s * rps + i * zr
        pltpu.sync_copy(acc_sh.at[pl.ds(r0, zr)], zbuf_v)
        pltpu.sync_copy(zbuf_v, out_hbm.at[c, pl.ds(r0, zr)])
        return carry

    lax.fori_loop(0, rps // zr, wout, 0)


@functools.lru_cache(maxsize=None)
def _make_propagate(F, nchunk, n_table):
    mesh = plsc.VectorSubcoreMesh(core_axis_name="c", subcore_axis_name="s")

    @functools.partial(
        pl.kernel,
        out_type=jax.ShapeDtypeStruct((NC, NACC_P, F), jnp.float32),
        mesh=mesh,
        compiler_params=pltpu.CompilerParams(use_tc_tiling_on_sc=False),
        scratch_types=[
            pltpu.VMEM((nchunk, K), jnp.int32),
            pltpu.VMEM((nchunk, K), jnp.int32),
            pltpu.VMEM((2, K, F), jnp.float32),
            pltpu.VMEM((ZR_P, F), jnp.float32),
            pltpu.VMEM_SHARED((NACC_P, F), jnp.float32),
            pltpu.SemaphoreType.DMA,
            pltpu.SemaphoreType.DMA,
        ],
    )
    def prop(tok_hbm, table_hbm, sidx_hbm, didx_hbm, out_hbm,
             sidx_v, didx_v, rows_v, zbuf_v, acc_sh, sem0, sem1):
        c = lax.axis_index("c")
        s = lax.axis_index("s")
        wid = s * NC + c

        pltpu.sync_copy(tok_hbm, zbuf_v.at[0, pl.ds(0, 16)])
        _zero_acc_slice(zbuf_v, acc_sh, s, F, RPS_P, ZR_P)
        pltpu.sync_copy(sidx_hbm.at[wid], sidx_v)
        pltpu.sync_copy(didx_hbm.at[wid], didx_v)
        plsc.subcore_barrier()

        sems = (sem0, sem1)
        pltpu.async_copy(table_hbm.at[sidx_v.at[0]], rows_v.at[0], sem0)
        pltpu.async_copy(table_hbm.at[sidx_v.at[1]], rows_v.at[1], sem1)

        def pair(i, carry):
            ci0 = 2 * i
            for b in range(2):
                ci = ci0 + b
                pltpu.make_async_copy(table_hbm.at[sidx_v.at[ci]],
                                      rows_v.at[b], sems[b]).wait()
                pltpu.sync_copy(rows_v.at[b], acc_sh.at[didx_v.at[ci]], add=True)
                nxt = jnp.minimum(ci + 2, nchunk - 1)

                @pl.when(ci + 2 < nchunk)
                def _():
                    pltpu.async_copy(table_hbm.at[sidx_v.at[nxt]],
                                     rows_v.at[b], sems[b])
            return carry

        lax.fori_loop(0, nchunk // 2, pair, 0)
        plsc.subcore_barrier()
        _writeout(zbuf_v, acc_sh, out_hbm, c, s, RPS_P, ZR_P)

    return prop


CF = 16


@functools.lru_cache(maxsize=None)
def _make_count(nchunk):
    mesh = plsc.VectorSubcoreMesh(core_axis_name="c", subcore_axis_name="s")

    @functools.partial(
        pl.kernel,
        out_type=jax.ShapeDtypeStruct((NC, NACC_C, CF), jnp.float32),
        mesh=mesh,
        compiler_params=pltpu.CompilerParams(use_tc_tiling_on_sc=False),
        scratch_types=[
            pltpu.VMEM((nchunk, K), jnp.int32),
            pltpu.VMEM((K, CF), jnp.float32),
            pltpu.VMEM((ZR_C, CF), jnp.float32),
            pltpu.VMEM_SHARED((NACC_C, CF), jnp.float32),
        ],
    )
    def cnt(tok_hbm, didx_hbm, out_hbm,
            didx_v, ones_v, zbuf_v, acc_sh):
        c = lax.axis_index("c")
        s = lax.axis_index("s")
        wid = s * NC + c

        pltpu.sync_copy(tok_hbm, zbuf_v.at[0, pl.ds(0, 16)])
        _zero_acc_slice(zbuf_v, acc_sh, s, CF, RPS_C, ZR_C)

        def orow(i, carry):
            ones_v[i, pl.ds(0, 16)] = jnp.ones((16,), jnp.float32)
            return carry

        lax.fori_loop(0, K, orow, 0)
        pltpu.sync_copy(didx_hbm.at[wid], didx_v)
        plsc.subcore_barrier()

        def chunk(ci, carry):
            pltpu.sync_copy(ones_v, acc_sh.at[didx_v.at[ci]], add=True)
            return carry

        lax.fori_loop(0, nchunk, chunk, 0)
        plsc.subcore_barrier()
        _writeout(zbuf_v, acc_sh, out_hbm, c, s, RPS_C, ZR_C)

    return cnt


def _shape_idx(flat, nchunk):
    return flat.reshape(NW, nchunk, K)


FP = 64


def _propagate(table, sidx, didx, nchunk, tok):
    assert table.shape[1] == FP
    parts = _make_propagate(FP, nchunk, table.shape[0])(tok, table, sidx, didx)
    return parts[0, :N] + parts[1, :N], parts[0, 0, :16]


def _propagate_wide(table, sidx, didx, nchunk, tok):
    halves = []
    for f0 in range(0, table.shape[1], FP):
        h, tok = _propagate(table[:, f0:f0 + FP], sidx, didx, nchunk, tok)
        halves.append(h)
    return jnp.concatenate(halves, axis=1) if len(halves) > 1 else halves[0], tok


def _count(didx, nchunk, tok):
    parts = _make_count(nchunk)(tok, didx)
    return parts[0, :N, 0] + parts[1, :N, 0], parts[0, 0, :16]


def _contrastive_body(z1_ref, z2_ref, out_ref):
    r = pl.program_id(0)
    z1 = z1_ref[...]
    inv_t = 1.0 / TEMPERATURE
    row_ids = r * ROW_BLK + jax.lax.broadcasted_iota(jnp.int32, (ROW_BLK, COL_BLK), 0)

    def step(c, carry):
        m, s, pos = carry
        z2c = z2_ref[pl.ds(c * COL_BLK, COL_BLK), :]
        sim = jax.lax.dot_general(z1, z2c, (((1,), (1,)), ((), ())),
                                  preferred_element_type=jnp.float32) * inv_t
        col_ids = c * COL_BLK + jax.lax.broadcasted_iota(jnp.int32, (ROW_BLK, COL_BLK), 1)
        diag = row_ids == col_ids
        pos = pos + jnp.sum(jnp.where(diag, sim, 0.0), axis=1, keepdims=True)
        simm = jnp.where(diag, -jnp.inf, sim)
        m_new = jnp.maximum(m, jnp.max(simm, axis=1, keepdims=True))
        s = s * jnp.exp(m - m_new) + jnp.sum(jnp.exp(simm - m_new), axis=1, keepdims=True)
        return m_new, s, pos

    m0 = jnp.full((ROW_BLK, 1), -jnp.inf, jnp.float32)
    s0 = jnp.zeros((ROW_BLK, 1), jnp.float32)
    p0 = jnp.zeros((ROW_BLK, 1), jnp.float32)
    m, s, pos = jax.lax.fori_loop(0, N // COL_BLK, step, (m0, s0, p0))
    out_ref[...] = pos - (m + jnp.log(s))


def _contrastive_loss_pallas(z1, z2):
    z1 = z1 / jnp.maximum(jnp.linalg.norm(z1, axis=1, keepdims=True), 1e-12)
    z2 = z2 / jnp.maximum(jnp.linalg.norm(z2, axis=1, keepdims=True), 1e-12)
    per_row = pl.pallas_call(
        _contrastive_body,
        grid=(N // ROW_BLK,),
        in_specs=[
            pl.BlockSpec((ROW_BLK, OUT_C), lambda r: (r, 0)),
            pl.BlockSpec((N, OUT_C), lambda r: (0, 0)),
        ],
        out_specs=pl.BlockSpec((ROW_BLK, 1), lambda r: (r, 0)),
        out_shape=jax.ShapeDtypeStruct((N, 1), jnp.float32),
    )(z1, z2)
    return -jnp.mean(per_row)


def _layer_norm(h, g, b, eps=1e-5):
    mu = jnp.mean(h, axis=-1, keepdims=True)
    var = jnp.mean((h - mu) ** 2, axis=-1, keepdims=True)
    return (h - mu) / jnp.sqrt(var + eps) * g + b


def kernel(x, edge_index, hyperedge_index, params):
    p = params
    key = jax.random.key(42)
    k1, k2, k3 = jax.random.split(key, 3)
    src, dst = edge_index[0], edge_index[1]
    mask = (jax.random.uniform(k1, x.shape) > MASK_RATIO).astype(x.dtype)
    x_bar = x * mask
    keep = (jax.random.uniform(k2, (src.shape[0],)) >= EDGE_DROP_RATIO) & (src <= dst)

    n = x.shape[0]
    loop = jnp.arange(n, dtype=src.dtype)
    s2 = jnp.concatenate([src, dst, loop])
    d2_raw = jnp.concatenate([dst, src, loop])
    valid = jnp.concatenate([keep, keep, jnp.ones((n,), bool)])
    spread = DUMMY + (jnp.arange(d2_raw.shape[0], dtype=jnp.int32) % DSPREAD)
    d2 = jnp.where(valid, d2_raw, spread)

    E2 = 2 * E + N
    nchunk_g = 2 * (-(-E2 // (2 * NW * K)))
    E2P = NW * K * nchunk_g
    s2f = _shape_idx(_pad_edges(s2, E2P, 0), nchunk_g)
    d2f = _shape_idx(_pad_didx(d2, E2P), nchunk_g)

    nid, hid_ = hyperedge_index[0], hyperedge_index[1]
    nchunk_h = 2 * (-(-HE // (2 * NW * K)))
    HEP = NW * K * nchunk_h
    nidf = _shape_idx(_pad_edges(nid, HEP, 0), nchunk_h)
    hidf = _shape_idx(_pad_didx(hid_, HEP), nchunk_h)
    nidf_d = _shape_idx(_pad_didx(nid, HEP), nchunk_h)

    tok = jnp.zeros((16,), jnp.float32)
    deg, tok = _count(d2f, nchunk_g, tok)
    Bc, tok = _count(hidf, nchunk_h, tok)
    Dc, tok = _count(nidf_d, nchunk_h, tok)
    dis = jnp.where(deg > 0, jax.lax.rsqrt(deg), 0.0)
    Binv = jnp.where(Bc > 0, 1.0 / Bc, 0.0)
    Dinv = jnp.where(Dc > 0, 1.0 / Dc, 0.0)

    def gcn_agg(h_in):
        nonlocal tok
        P, tok = _propagate_wide(dis[:, None] * h_in, s2f, d2f, nchunk_g, tok)
        return dis[:, None] * P

    def lnrelu(z, g, b):
        return jax.nn.relu(_layer_norm(z, g, b))

    Q1 = gcn_agg(x_bar)

    h = lnrelu(Q1 @ p['ex_gcn1_W'] + p['ex_gcn1_b'], p['ex_ln1_g'], p['ex_ln1_b'])
    h = lnrelu(gcn_agg(h) @ p['ex_gcn2_W'] + p['ex_gcn2_b'], p['ex_ln1_g'], p['ex_ln1_b'])
    h_x = jax.nn.relu(h @ p['ex_p1_W'] + p['ex_p1_b']) @ p['ex_p2_W'] + p['ex_p2_b']

    h1 = lnrelu(Q1 @ p['ey_gcn1_W'] + p['ey_gcn1_b'], p['ey_ln1_g'], p['ey_ln1_b'])
    h1 = lnrelu(gcn_agg(h1) @ p['ey_gcn2_W'] + p['ey_gcn2_b'], p['ey_ln1_g'], p['ey_ln1_b'])

    def hgc_agg(h_in):
        nonlocal tok
        M1, tok = _propagate_wide(h_in, nidf, hidf, nchunk_h, tok)
        P2, tok = _propagate_wide(Binv[:, None] * M1, hidf, nidf_d, nchunk_h, tok)
        return Dinv[:, None] * P2

    h2 = lnrelu(hgc_agg(x_bar) @ p['ey_hgc1_W'] + p['ey_hgc1_b'], p['ey_ln2_g'], p['ey_ln2_b'])
    h2 = lnrelu(hgc_agg(h2) @ p['ey_hgc2_W'] + p['ey_hgc2_b'], p['ey_ln2_g'], p['ey_ln2_b'])
    hy = (h1 + h2) / 2.0
    h_y = jax.nn.relu(hy @ p['ey_p1_W'] + p['ey_p1_b']) @ p['ey_p2_W'] + p['ey_p2_b']

    loss_c = _contrastive_loss_pallas(h_x, h_y)

    t = 10
    beta = jnp.linspace(BETA_START, BETA_END, T_DIFFUSION)
    alpha_cum = jnp.cumprod(1.0 - beta)
    sa = jnp.sqrt(alpha_cum[t])
    so = jnp.sqrt(1.0 - alpha_cum[t])
    noise = jax.random.normal(k3, h_x.shape, dtype=h_x.dtype)
    h_noisy = sa * h_x + so * noise
    t_in = jnp.array([[t / T_DIFFUSION]], dtype=h_x.dtype)
    t_emb = jax.nn.relu(t_in @ p['dn_t1_W'] + p['dn_t1_b']) @ p['dn_t2_W'] + p['dn_t2_b']
    hn = h_noisy + t_emb
    hn = lnrelu(gcn_agg(hn) @ p['dn_c1_W'] + p['dn_c1_b'], p['dn_ln_g'], p['dn_ln_b'])
    h_hat = gcn_agg(hn) @ p['dn_c2_W'] + p['dn_c2_b']
    loss_g = jnp.mean((h_hat - h_x) ** 2)
    loss = GAMMA * loss_c + (1.0 - GAMMA) * loss_g
    return loss, jax.lax.stop_gradient(h_x)
